# baseline (speedup 1.0000x reference)
import jax
import jax.numpy as jnp
import numpy as np
from jax.experimental import pallas as pl
from jax.experimental.pallas import tpu as pltpu

N, IN_DIM, OUT_DIM, HEADS = 1024, 128, 64, 2
C = OUT_DIM // HEADS
BD = 128
GRID = N // BD
LOG2E = float(np.log2(np.e))


def _gat_kernel(A_ref, X_ref, W_ref, att_src_ref, att_dst_ref, bias_ref,
                o_ref, haug0_ref, haug1_ref, asrc_ref, adst_ref):
    j = pl.program_id(0)

    @pl.when(j == 0)
    def _prologue():
        h = jnp.dot(X_ref[...], W_ref[...],
                    preferred_element_type=jnp.float32)
        hs = h * att_src_ref[...]
        hd = h * att_dst_ref[...]
        ones = jnp.ones((N, 1), dtype=jnp.float32)
        haug0_ref[...] = jnp.concatenate([h[:, :C], ones], axis=1)
        haug1_ref[...] = jnp.concatenate([h[:, C:], ones], axis=1)
        asrc_ref[...] = jnp.stack(
            [jnp.sum(hs[:, :C], axis=1), jnp.sum(hs[:, C:], axis=1)],
            axis=1) * LOG2E
        adst_ref[...] = jnp.stack(
            [jnp.sum(hd[:, :C], axis=1), jnp.sum(hd[:, C:], axis=1)],
            axis=1) * LOG2E

    A = A_ref[...]
    outs = []
    for head, haug in ((0, haug0_ref), (1, haug1_ref)):
        adst_tile = adst_ref[pl.ds(j * BD, BD), head]
        x = asrc_ref[:, head][:, None] + adst_tile[None, :]
        x = jnp.maximum(x, 0.2 * x)
        E = A * jnp.exp2(x)
        r = jax.lax.dot_general(
            E, haug[...], (((0,), (0,)), ((), ())),
            preferred_element_type=jnp.float32)
        outs.append(r[:, :C] / (r[:, C:] + 1e-16))
    out = jnp.concatenate(outs, axis=1) + bias_ref[...]
    o_ref[...] = jnp.maximum(out, 0.0)


@jax.jit
def kernel(A, X, W, att_src, att_dst, bias):
    att_src2 = att_src.reshape(1, HEADS * C)
    att_dst2 = att_dst.reshape(1, HEADS * C)
    bias2 = bias.reshape(1, HEADS * C)
    return pl.pallas_call(
        _gat_kernel,
        grid=(GRID,),
        in_specs=[
            pl.BlockSpec((N, BD), lambda j: (0, j)),
            pl.BlockSpec((N, IN_DIM), lambda j: (0, 0)),
            pl.BlockSpec((IN_DIM, HEADS * C), lambda j: (0, 0)),
            pl.BlockSpec((1, HEADS * C), lambda j: (0, 0)),
            pl.BlockSpec((1, HEADS * C), lambda j: (0, 0)),
            pl.BlockSpec((1, HEADS * C), lambda j: (0, 0)),
        ],
        out_specs=pl.BlockSpec((BD, HEADS * C), lambda j: (j, 0)),
        out_shape=jax.ShapeDtypeStruct((N, HEADS * C), jnp.float32),
        scratch_shapes=[
            pltpu.VMEM((N, C + 1), jnp.float32),
            pltpu.VMEM((N, C + 1), jnp.float32),
            pltpu.VMEM((N, HEADS), jnp.float32),
            pltpu.VMEM((N, HEADS), jnp.float32),
        ],
    )(A, X, W, att_src2, att_dst2, bias2)


# speedup vs baseline: 1.4049x; 1.4049x over previous
import jax
import jax.numpy as jnp
import numpy as np
from jax.experimental import pallas as pl

N, IN_DIM, OUT_DIM, HEADS = 1024, 128, 64, 2
C = OUT_DIM // HEADS
LOG2E = float(np.log2(np.e))


def _gat_kernel(A_ref, X_ref, W_ref, att_src_ref, att_dst_ref, bias_ref,
                o_ref):
    h = jnp.dot(X_ref[...], W_ref[...],
                preferred_element_type=jnp.float32)
    hs = h * att_src_ref[...]
    hd = h * att_dst_ref[...]
    ones = jnp.ones((N, 1), dtype=jnp.float32)
    A = A_ref[...]
    outs = []
    for head in range(HEADS):
        sl = slice(head * C, (head + 1) * C)
        a_src = jnp.sum(hs[:, sl], axis=1) * LOG2E
        a_dst = jnp.sum(hd[:, sl], axis=1) * LOG2E
        x = a_src[:, None] + a_dst[None, :]
        x = jnp.maximum(x, 0.2 * x)
        E = A * jnp.exp2(x)
        haug = jnp.concatenate([h[:, sl], ones], axis=1)
        r = jax.lax.dot_general(
            E, haug, (((0,), (0,)), ((), ())),
            preferred_element_type=jnp.float32)
        outs.append(r[:, :C] / (r[:, C:] + 1e-16))
    out = jnp.concatenate(outs, axis=1) + bias_ref[...]
    o_ref[...] = jnp.maximum(out, 0.0)


@jax.jit
def kernel(A, X, W, att_src, att_dst, bias):
    att_src2 = att_src.reshape(1, HEADS * C)
    att_dst2 = att_dst.reshape(1, HEADS * C)
    bias2 = bias.reshape(1, HEADS * C)
    return pl.pallas_call(
        _gat_kernel,
        out_shape=jax.ShapeDtypeStruct((N, HEADS * C), jnp.float32),
    )(A, X, W, att_src2, att_dst2, bias2)
